# baseline (speedup 1.0000x reference)
import functools

import jax
import jax.numpy as jnp
from jax.experimental import pallas as pl
from jax.experimental.pallas import tpu as pltpu

_BANK_ROWS = 65536
_BATCH = 4096
_DIM = 128
_NSTRIPE = 8


def _body(output_hbm, bank_hbm, out_hbm, osem, *sems):
    rows = _BANK_ROWS - _BATCH
    stripe = rows // _NSTRIPE
    copies = [
        pltpu.make_async_copy(
            output_hbm.at[pl.ds(0, _BATCH)],
            out_hbm.at[pl.ds(0, _BATCH)],
            osem,
        )
    ]
    for s in range(_NSTRIPE):
        base = _BATCH + s * stripe
        copies.append(
            pltpu.make_async_copy(
                bank_hbm.at[pl.ds(base, stripe)],
                out_hbm.at[pl.ds(base, stripe)],
                sems[s],
            )
        )
    for c in copies:
        c.start()
    for c in copies:
        c.wait()


@functools.cache
def _bank_update_kernel():
    return pl.pallas_call(
        _body,
        in_specs=[
            pl.BlockSpec(memory_space=pl.ANY),
            pl.BlockSpec(memory_space=pl.ANY),
        ],
        out_specs=pl.BlockSpec(memory_space=pl.ANY),
        out_shape=jax.ShapeDtypeStruct((_BANK_ROWS, _DIM), jnp.float32),
        scratch_shapes=[pltpu.SemaphoreType.DMA] * (_NSTRIPE + 1),
    )


def kernel(output, bank):
    new_bank = _bank_update_kernel()(output, bank)
    return (output, bank, new_bank)


# speedup vs baseline: 17.8621x; 17.8621x over previous
import functools

import jax
import jax.numpy as jnp
from jax.experimental import pallas as pl
from jax.experimental.pallas import tpu as pltpu

_BANK_ROWS = 65536
_BATCH = 4096
_DIM = 128
_BLOCK = 2048


def _body(output_ref, bank_ref, out_ref):
    i = pl.program_id(0)
    nb_out = _BATCH // _BLOCK

    @pl.when(i < nb_out)
    def _():
        out_ref[...] = output_ref[...]

    @pl.when(i >= nb_out)
    def _():
        out_ref[...] = bank_ref[...]


@functools.cache
def _bank_update_kernel():
    grid = _BANK_ROWS // _BLOCK
    nb_out = _BATCH // _BLOCK
    return pl.pallas_call(
        _body,
        grid=(grid,),
        in_specs=[
            pl.BlockSpec((_BLOCK, _DIM), lambda i: (jnp.minimum(i, _BATCH // _BLOCK - 1), 0)),
            pl.BlockSpec((_BLOCK, _DIM), lambda i: (jnp.maximum(i, _BATCH // _BLOCK), 0)),
        ],
        out_specs=pl.BlockSpec((_BLOCK, _DIM), lambda i: (i, 0)),
        out_shape=jax.ShapeDtypeStruct((_BANK_ROWS, _DIM), jnp.float32),
    )


def kernel(output, bank):
    new_bank = _bank_update_kernel()(output, bank)
    return (output, bank, new_bank)


# speedup vs baseline: 28.0308x; 1.5693x over previous
import functools

import jax
import jax.numpy as jnp
from jax.experimental import pallas as pl
from jax.experimental.pallas import tpu as pltpu

_BANK_ROWS = 65536
_BATCH = 4096
_DIM = 128
_BLOCK = 4096


def _body(output_ref, bank_ref, out_clone_ref, bank_clone_ref, new_bank_ref):
    i = pl.program_id(0)
    b = bank_ref[...]
    bank_clone_ref[...] = b

    @pl.when(i == 0)
    def _():
        o = output_ref[...]
        out_clone_ref[...] = o
        new_bank_ref[...] = o

    @pl.when(i != 0)
    def _():
        new_bank_ref[...] = b


@functools.cache
def _bank_update_kernel():
    grid = _BANK_ROWS // _BLOCK
    return pl.pallas_call(
        _body,
        grid=(grid,),
        in_specs=[
            pl.BlockSpec((_BATCH, _DIM), lambda i: (0, 0)),
            pl.BlockSpec((_BLOCK, _DIM), lambda i: (i, 0)),
        ],
        out_specs=[
            pl.BlockSpec((_BATCH, _DIM), lambda i: (0, 0)),
            pl.BlockSpec((_BLOCK, _DIM), lambda i: (i, 0)),
            pl.BlockSpec((_BLOCK, _DIM), lambda i: (i, 0)),
        ],
        out_shape=[
            jax.ShapeDtypeStruct((_BATCH, _DIM), jnp.float32),
            jax.ShapeDtypeStruct((_BANK_ROWS, _DIM), jnp.float32),
            jax.ShapeDtypeStruct((_BANK_ROWS, _DIM), jnp.float32),
        ],
    )


def kernel(output, bank):
    out_clone, bank_clone, new_bank = _bank_update_kernel()(output, bank)
    return (out_clone, bank_clone, new_bank)


# speedup vs baseline: 30.1928x; 1.0771x over previous
import functools

import jax
import jax.numpy as jnp
from jax.experimental import pallas as pl
from jax.experimental.pallas import tpu as pltpu

_BANK_ROWS = 65536
_BATCH = 4096
_DIM = 128
_BLOCK = 8192


def _body(output_ref, bank_ref, out_clone_ref, bank_clone_ref, new_bank_ref):
    i = pl.program_id(0)
    b = bank_ref[...]
    bank_clone_ref[...] = b

    @pl.when(i == 0)
    def _():
        o = output_ref[...]
        out_clone_ref[...] = o
        new_bank_ref[0:_BATCH] = o
        new_bank_ref[_BATCH:] = b[_BATCH:]

    @pl.when(i != 0)
    def _():
        new_bank_ref[...] = b


@functools.cache
def _bank_update_kernel():
    grid = _BANK_ROWS // _BLOCK
    return pl.pallas_call(
        _body,
        grid=(grid,),
        in_specs=[
            pl.BlockSpec((_BATCH, _DIM), lambda i: (0, 0)),
            pl.BlockSpec((_BLOCK, _DIM), lambda i: (i, 0)),
        ],
        out_specs=[
            pl.BlockSpec((_BATCH, _DIM), lambda i: (0, 0)),
            pl.BlockSpec((_BLOCK, _DIM), lambda i: (i, 0)),
            pl.BlockSpec((_BLOCK, _DIM), lambda i: (i, 0)),
        ],
        out_shape=[
            jax.ShapeDtypeStruct((_BATCH, _DIM), jnp.float32),
            jax.ShapeDtypeStruct((_BANK_ROWS, _DIM), jnp.float32),
            jax.ShapeDtypeStruct((_BANK_ROWS, _DIM), jnp.float32),
        ],
    )


def kernel(output, bank):
    out_clone, bank_clone, new_bank = _bank_update_kernel()(output, bank)
    return (out_clone, bank_clone, new_bank)
